# baseline (speedup 1.0000x reference)
import functools

import jax
import jax.numpy as jnp
from jax import lax
from jax.experimental import pallas as pl
from jax.experimental.pallas import tpu as pltpu
from jax.experimental.pallas import tpu_sc as plsc

B = 16
H = 512
W = 512
NPIX = H * W
HALF = H // 2
NBINS = 256
LANES = 16
ROWS = 32
CH = ROWS * W
NCHUNK = HALF // ROWS
SMOOTH = 1e-6
EPS = 1e-6


def _sc_hist_kernel(e_hbm, o_hbm, out_hbm, ebuf0, ebuf1, obuf0, obuf1,
                    hist, hist256, sem0, sem1, sem2, sem3):
    c = lax.axis_index("c")
    s = lax.axis_index("s")
    lanebase = lax.iota(jnp.int32, LANES) * NBINS
    zero16 = jnp.zeros((LANES,), jnp.float32)
    one16 = jnp.ones((LANES,), jnp.float32)

    for i in range(2 * LANES * NBINS // LANES):
        hist[pl.ds(i * LANES, LANES)] = zero16

    ebufs = (ebuf0, ebuf1)
    obufs = (obuf0, obuf1)
    esems = (sem0, sem1)
    osems = (sem2, sem3)

    def start(ci, k):
        r0 = c * HALF + ci * ROWS
        pltpu.async_copy(e_hbm.at[s, pl.ds(r0, ROWS), :], ebufs[k], esems[k])
        pltpu.async_copy(o_hbm.at[s, pl.ds(r0, ROWS), :], obufs[k], osems[k])

    start(0, 0)
    for ci in range(NCHUNK):
        k = ci % 2
        if ci + 1 < NCHUNK:
            start(ci + 1, (ci + 1) % 2)
        pltpu.make_async_copy(e_hbm.at[0, pl.ds(0, ROWS), :], ebufs[k],
                              esems[k]).wait()
        pltpu.make_async_copy(o_hbm.at[0, pl.ds(0, ROWS), :], obufs[k],
                              osems[k]).wait()

        @plsc.parallel_loop(0, W // LANES, 1, unroll=2)
        def _(j, k=k):
            for tt in range(2):
                buf = (ebufs if tt == 0 else obufs)[k]
                tbase = tt * (LANES * NBINS)
                for r in range(ROWS):
                    x = buf[r, pl.ds(j * LANES, LANES)]
                    idx = (x * 256.0).astype(jnp.int32)
                    w = jnp.where(x > 0.0, one16, zero16)
                    plsc.addupdate_scatter(hist, [tbase + lanebase + idx], w)

    for tt in range(2):
        tbase = tt * (LANES * NBINS)
        for j in range(NBINS // LANES):
            acc = zero16
            for l in range(LANES):
                acc = acc + hist[pl.ds(tbase + l * NBINS + j * LANES, LANES)]
            hist256[pl.ds(j * LANES, LANES)] = acc
        pltpu.sync_copy(hist256, out_hbm.at[c, tt, s])


def _sc_histograms(e3, o3):
    mesh = plsc.VectorSubcoreMesh(core_axis_name="c", subcore_axis_name="s")
    kern = functools.partial(
        pl.kernel,
        out_type=jax.ShapeDtypeStruct((2, 2, B, NBINS), jnp.float32),
        mesh=mesh,
        compiler_params=pltpu.CompilerParams(needs_layout_passes=False),
        scratch_types=[
            pltpu.VMEM((ROWS, W), jnp.float32),
            pltpu.VMEM((ROWS, W), jnp.float32),
            pltpu.VMEM((ROWS, W), jnp.float32),
            pltpu.VMEM((ROWS, W), jnp.float32),
            pltpu.VMEM((2 * LANES * NBINS,), jnp.float32),
            pltpu.VMEM((NBINS,), jnp.float32),
            pltpu.SemaphoreType.DMA,
            pltpu.SemaphoreType.DMA,
            pltpu.SemaphoreType.DMA,
            pltpu.SemaphoreType.DMA,
        ],
    )(_sc_hist_kernel)
    return kern(e3, o3)


def _tc_hist_loss_kernel(hh_ref, out_ref):
    he = hh_ref[0, 0] + hh_ref[1, 0]
    ho = hh_ref[0, 1] + hh_ref[1, 1]
    se = jnp.sum(he, axis=1, keepdims=True)
    so = jnp.sum(ho, axis=1, keepdims=True)
    hen = (he + SMOOTH) / (se + SMOOTH)
    hon = (ho + SMOOTH) / (so + SMOOTH)
    d = hen - hon
    out_ref[0] = jnp.sum(d * d) / float(B * NBINS) / float(NBINS)


def _tc_hist_loss(hh):
    return pl.pallas_call(
        _tc_hist_loss_kernel,
        out_specs=pl.BlockSpec(memory_space=pltpu.SMEM),
        out_shape=jax.ShapeDtypeStruct((1,), jnp.float32),
    )(hh)


def _lap_abs_sum(a):
    zr = jnp.zeros((1, W), jnp.float32)
    zc = jnp.zeros((H, 1), jnp.float32)
    up = jnp.concatenate([zr, a[:-1, :]], axis=0)
    dn = jnp.concatenate([a[1:, :], zr], axis=0)
    lf = jnp.concatenate([zc, a[:, :-1]], axis=1)
    rt = jnp.concatenate([a[:, 1:], zc], axis=1)
    return jnp.sum(jnp.abs(up + dn + lf + rt - 4.0 * a))


def _tc_dense_kernel(e_ref, o_ref, out_ref):
    bidx = pl.program_id(0)
    a = e_ref[0]
    ao = o_ref[0]

    l1 = jnp.sum(jnp.abs(a - ao))
    lape = _lap_abs_sum(a)
    lapo = _lap_abs_sum(ao)
    dxs = jnp.sum(jnp.abs(a[1:, :] - a[:-1, :]))
    dys = jnp.sum(jnp.abs(a[:, 1:] - a[:, :-1]))

    @pl.when(bidx == 0)
    def _():
        for i in range(8):
            out_ref[i] = 0.0

    out_ref[0] += l1
    out_ref[1] += lape
    out_ref[2] += lapo
    out_ref[3] += dxs
    out_ref[4] += dys


def _tc_dense_sums(e3, o3):
    return pl.pallas_call(
        _tc_dense_kernel,
        grid=(B,),
        in_specs=[
            pl.BlockSpec((1, H, W), lambda b: (b, 0, 0)),
            pl.BlockSpec((1, H, W), lambda b: (b, 0, 0)),
        ],
        out_specs=pl.BlockSpec(memory_space=pltpu.SMEM),
        out_shape=jax.ShapeDtypeStruct((8,), jnp.float32),
    )(e3, o3)


def kernel(enhanced_y, original_y):
    e3 = enhanced_y.reshape(B, H, W)
    o3 = original_y.reshape(B, H, W)

    hh = _sc_histograms(e3, o3)
    sums = _tc_dense_sums(e3, o3)
    hist_loss = _tc_hist_loss(hh)[0]

    n = float(B * NPIX)
    l1 = sums[0] / n
    ce = sums[1] / n
    co = sums[2] / n
    cont = jnp.abs(ce - co) / (co + EPS)
    nd = float(B * (H - 1) * W)
    smooth = sums[3] / nd + sums[4] / nd
    return l1 + 0.1 * hist_loss + 0.1 * cont + 0.01 * smooth
